# Initial kernel scaffold; baseline (speedup 1.0000x reference)
import jax
import jax.numpy as jnp
from jax.experimental import pallas as pl


def kernel(x, W1f, W1d, g1, b1, W2f, W2d, g2, b2, W3f, W3d, g3, b3, W4f, W4d, g4, b4, W5f, W5d, g5, b5):
    raise NotImplementedError("write your pallas kernel here")


# speedup vs baseline: 1.2279x; 1.2279x over previous
import functools

import jax
import jax.numpy as jnp
from jax import lax
from jax.experimental import pallas as pl
from jax.experimental.pallas import tpu as pltpu

EPS = 1e-6
B = 2
N = 1024
K = 20
NBLK = 256
NBS = N // NBLK


def _knn_body(x_ref, idx_ref, *, n, k):
    xb = x_ref[0]
    xbb = xb.astype(jnp.bfloat16)
    ip = lax.dot_general(xbb, xbb, (((0,), (0,)), ((), ())),
                         preferred_element_type=jnp.float32)
    inner = -2.0 * ip
    xx = jnp.sum(xb * xb, axis=0)
    pd = (-xx[None, :]) - inner - xx[:, None]
    iot = lax.broadcasted_iota(jnp.int32, (n, n), 1)
    for t in range(k):
        m = jnp.max(pd, axis=1, keepdims=True)
        am = jnp.min(jnp.where(pd >= m, iot, n), axis=1).astype(jnp.int32)
        idx_ref[0, t, 0, :] = am
        pd = jnp.where(iot == am[:, None], -jnp.float32(3e38), pd)


def _knn(xt, cn):
    return pl.pallas_call(
        functools.partial(_knn_body, n=N, k=K),
        grid=(B,),
        in_specs=[pl.BlockSpec((1, cn, N), lambda b: (b, 0, 0))],
        out_specs=pl.BlockSpec((1, K, 1, N), lambda b: (b, 0, 0, 0)),
        out_shape=jax.ShapeDtypeStruct((B, K, 1, N), jnp.int32),
    )(xt)


def _gather_body(idx_ref, x_ref, g_ref, *, n):
    idx = idx_ref[0, 0, 0, :]
    iot = lax.broadcasted_iota(jnp.int32, (n, idx.shape[0]), 0)
    oh = jnp.where(iot == idx[None, :], 1.0, 0.0).astype(jnp.float32)
    g_ref[0, 0] = lax.dot_general(oh, x_ref[0], (((0,), (0,)), ((), ())),
                                  preferred_element_type=jnp.float32,
                                  precision=lax.Precision.HIGHEST)


def _gather(idx, xr, crow):
    npb = NBLK
    return pl.pallas_call(
        functools.partial(_gather_body, n=N),
        grid=(B, K, N // npb),
        in_specs=[
            pl.BlockSpec((1, 1, 1, npb), lambda b, j, p: (b, j, 0, p)),
            pl.BlockSpec((1, N, crow), lambda b, j, p: (b, 0, 0)),
        ],
        out_specs=pl.BlockSpec((1, 1, npb, crow), lambda b, j, p: (b, j, p, 0)),
        out_shape=jax.ShapeDtypeStruct((B, K, N, crow), jnp.float32),
    )(idx, xr)


def _pmm(g, xc, w):
    outs = []
    for d in range(3):
        gd = g[:, d, :]
        cd = xc[:, d, :]
        feat = jnp.concatenate([gd - cd, cd], axis=1).astype(jnp.bfloat16)
        outs.append(lax.dot_general(feat, w, (((1,), (0,)), ((), ())),
                                    preferred_element_type=jnp.float32))
    return outs


def _stats_body(g_ref, xc_ref, wf_ref, s_ref, *, cnt):
    ph = pl.program_id(0)
    first = ((pl.program_id(1) == 0) & (pl.program_id(2) == 0)
             & (pl.program_id(3) == 0))

    @pl.when((ph == 0) & first)
    def _():
        s_ref[...] = jnp.zeros_like(s_ref)

    p = _pmm(g_ref[0, 0], xc_ref[0], wf_ref[...])
    nsq = p[0] * p[0] + p[1] * p[1] + p[2] * p[2]
    norm = jnp.sqrt(nsq) + EPS

    @pl.when(ph == 0)
    def _():
        s_ref[0, :] += jnp.sum(norm, axis=0)

    @pl.when(ph == 1)
    def _():
        mean = s_ref[0, :] / cnt
        dev = norm - mean[None, :]
        s_ref[1, :] += jnp.sum(dev * dev, axis=0)


def _stats(g, xr4, wf, c8, cout):
    return pl.pallas_call(
        functools.partial(_stats_body, cnt=B * N * K),
        grid=(2, B, K, NBS),
        in_specs=[
            pl.BlockSpec((1, 1, NBLK, 3, c8),
                         lambda ph, b, j, nb: (b, j, nb, 0, 0)),
            pl.BlockSpec((1, NBLK, 3, c8), lambda ph, b, j, nb: (b, nb, 0, 0)),
            pl.BlockSpec((2 * c8, cout), lambda ph, b, j, nb: (0, 0)),
        ],
        out_specs=pl.BlockSpec((2, cout), lambda ph, b, j, nb: (0, 0)),
        out_shape=jax.ShapeDtypeStruct((2, cout), jnp.float32),
    )(g, xr4, wf)


def _apply_body(s_ref, gam_ref, bet_ref, g_ref, xc_ref, wf_ref, wd_ref, o_ref,
                *, cnt):
    j = pl.program_id(2)
    mean = s_ref[0, :] / cnt
    var = s_ref[1, :] / cnt
    istd = jnp.sqrt(var + 1e-5)
    gam = gam_ref[0]
    bet = bet_ref[0]

    g = g_ref[0, 0]
    xc = xc_ref[0]
    p = _pmm(g, xc, wf_ref[...])
    dd = _pmm(g, xc, wd_ref[...])
    nsq = p[0] * p[0] + p[1] * p[1] + p[2] * p[2]
    norm = jnp.sqrt(nsq) + EPS
    nbn = (norm - mean[None, :]) / istd[None, :] * gam[None, :] + bet[None, :]
    pr = [pi / norm * nbn for pi in p]
    dot = pr[0] * dd[0] + pr[1] * dd[1] + pr[2] * dd[2]
    dnsq = dd[0] * dd[0] + dd[1] * dd[1] + dd[2] * dd[2]
    coef = jnp.where(dot >= 0.0, 0.0, dot / (dnsq + EPS))

    @pl.when(j == 0)
    def _():
        o_ref[...] = jnp.zeros_like(o_ref)

    for d in range(3):
        od = pr[d] - coef * dd[d]
        o_ref[0, :, d, :] += od

    @pl.when(j == K - 1)
    def _():
        o_ref[...] *= (1.0 / K)


def _apply(stats, gam, bet, g, xr4, wf, wd, c8, cout):
    return pl.pallas_call(
        functools.partial(_apply_body, cnt=B * N * K),
        grid=(B, NBS, K),
        in_specs=[
            pl.BlockSpec((2, cout), lambda b, nb, j: (0, 0)),
            pl.BlockSpec((1, cout), lambda b, nb, j: (0, 0)),
            pl.BlockSpec((1, cout), lambda b, nb, j: (0, 0)),
            pl.BlockSpec((1, 1, NBLK, 3, c8), lambda b, nb, j: (b, j, nb, 0, 0)),
            pl.BlockSpec((1, NBLK, 3, c8), lambda b, nb, j: (b, nb, 0, 0)),
            pl.BlockSpec((2 * c8, cout), lambda b, nb, j: (0, 0)),
            pl.BlockSpec((2 * c8, cout), lambda b, nb, j: (0, 0)),
        ],
        out_specs=pl.BlockSpec((1, NBLK, 3, cout), lambda b, nb, j: (b, nb, 0, 0)),
        out_shape=jax.ShapeDtypeStruct((B, N, 3, cout), jnp.float32),
    )(stats, gam, bet, g, xr4, wf, wd)


def _mm5_body(x_ref, w_ref, p_ref):
    xb = x_ref[0].astype(jnp.bfloat16)
    p_ref[0] = lax.dot_general(xb, w_ref[...], (((1,), (0,)), ((), ())),
                               preferred_element_type=jnp.float32)


def _mm5(xr, w5, crow, segw):
    return pl.pallas_call(
        _mm5_body,
        grid=(B, 6),
        in_specs=[
            pl.BlockSpec((1, N, crow), lambda b, s: (b, 0, 0)),
            pl.BlockSpec((crow, segw), lambda b, s: (0, s)),
        ],
        out_specs=pl.BlockSpec((1, N, segw), lambda b, s: (b, 0, s)),
        out_shape=jax.ShapeDtypeStruct((B, N, 6 * segw), jnp.float32),
    )(xr, w5)


def _stats5_body(g_ref, s_ref, *, segw, cnt):
    ph = pl.program_id(0)
    first = (pl.program_id(1) == 0) & (pl.program_id(2) == 0)

    @pl.when((ph == 0) & first)
    def _():
        s_ref[...] = jnp.zeros_like(s_ref)

    g = g_ref[0]
    nsq = None
    for d in range(3):
        p = g[:, d * segw:(d + 1) * segw]
        nsq = p * p if nsq is None else nsq + p * p
    norm = jnp.sqrt(nsq) + EPS

    @pl.when(ph == 0)
    def _():
        s_ref[0, :] += jnp.sum(norm, axis=0)

    @pl.when(ph == 1)
    def _():
        mean = s_ref[0, :] / cnt
        dev = norm - mean[None, :]
        s_ref[1, :] += jnp.sum(dev * dev, axis=0)


def _stats5(p5, segw):
    return pl.pallas_call(
        functools.partial(_stats5_body, segw=segw, cnt=B * N),
        grid=(2, B, NBS),
        in_specs=[pl.BlockSpec((1, NBLK, 6 * segw),
                               lambda ph, b, nb: (b, nb, 0))],
        out_specs=pl.BlockSpec((2, segw), lambda ph, b, nb: (0, 0)),
        out_shape=jax.ShapeDtypeStruct((2, segw), jnp.float32),
    )(p5)


def _apply5_body(s_ref, gam_ref, bet_ref, g_ref, o_ref, *, segw, cnt):
    nb = pl.program_id(1)
    mean = s_ref[0, :] / cnt
    var = s_ref[1, :] / cnt
    istd = jnp.sqrt(var + 1e-5)
    gam = gam_ref[0]
    bet = bet_ref[0]

    g = g_ref[0]
    p = [g[:, d * segw:(d + 1) * segw] for d in range(3)]
    dd = [g[:, (3 + d) * segw:(4 + d) * segw] for d in range(3)]
    nsq = p[0] * p[0] + p[1] * p[1] + p[2] * p[2]
    norm = jnp.sqrt(nsq) + EPS
    nbn = (norm - mean[None, :]) / istd[None, :] * gam[None, :] + bet[None, :]
    pr = [pi / norm * nbn for pi in p]
    dot = pr[0] * dd[0] + pr[1] * dd[1] + pr[2] * dd[2]
    dnsq = dd[0] * dd[0] + dd[1] * dd[1] + dd[2] * dd[2]
    coef = jnp.where(dot >= 0.0, 0.0, dot / (dnsq + EPS))

    @pl.when(nb == 0)
    def _():
        o_ref[...] = jnp.zeros_like(o_ref)

    for d in range(3):
        od = pr[d] - coef * dd[d]
        o_ref[0, d, :] += jnp.sum(od, axis=0)

    @pl.when(nb == NBS - 1)
    def _():
        o_ref[...] *= (1.0 / N)


def _apply5(stats, gam, bet, p5, segw):
    return pl.pallas_call(
        functools.partial(_apply5_body, segw=segw, cnt=B * N),
        grid=(B, NBS),
        in_specs=[
            pl.BlockSpec((2, segw), lambda b, nb: (0, 0)),
            pl.BlockSpec((1, segw), lambda b, nb: (0, 0)),
            pl.BlockSpec((1, segw), lambda b, nb: (0, 0)),
            pl.BlockSpec((1, NBLK, 6 * segw), lambda b, nb: (b, nb, 0)),
        ],
        out_specs=pl.BlockSpec((1, 3, segw), lambda b, nb: (b, 0, 0)),
        out_shape=jax.ShapeDtypeStruct((B, 3, segw), jnp.float32),
    )(stats, gam, bet, p5)


def _blockdiag(wt, crow, c, segw):
    cout = wt.shape[1]
    m = jnp.zeros((crow, 3 * segw), dtype=wt.dtype)
    for d in range(3):
        m = m.at[d * c:(d + 1) * c, d * segw:d * segw + cout].set(wt)
    return m


def _prep_w2(w, c8):
    c = w.shape[1] // 2
    wa = jnp.pad(w[:, :c].T, ((0, c8 - c), (0, 0)))
    wb = jnp.pad(w[:, c:].T, ((0, c8 - c), (0, 0)))
    return jnp.concatenate([wa, wb], axis=0).astype(jnp.bfloat16)


def _conv_layer(xr4, xt, wf, wd, gam, bet, cout):
    c8 = xr4.shape[-1]
    crow = 3 * c8
    idx = _knn(xt, xt.shape[1])
    wfp = _prep_w2(wf, c8)
    wdp = _prep_w2(wd, c8)
    g = _gather(idx, xr4.reshape(B, N, crow), crow).reshape(B, K, N, 3, c8)
    stats = _stats(g, xr4, wfp, c8, cout)
    out = _apply(stats, gam.reshape(1, cout), bet.reshape(1, cout),
                 g, xr4, wfp, wdp, c8, cout)
    xt_next = jnp.transpose(out, (0, 3, 2, 1)).reshape(B, cout * 3, N)
    return out, xt_next


def kernel(x, W1f, W1d, g1, b1, W2f, W2d, g2, b2, W3f, W3d, g3, b3,
           W4f, W4d, g4, b4, W5f, W5d, g5, b5):
    xr0 = jnp.pad(jnp.transpose(x, (0, 2, 1))[..., None],
                  ((0, 0), (0, 0), (0, 0), (0, 7)))
    x1, t1 = _conv_layer(xr0, x, W1f, W1d, g1, b1, 64)
    x2, t2 = _conv_layer(x1, t1, W2f, W2d, g2, b2, 64)
    x3, t3 = _conv_layer(x2, t2, W3f, W3d, g3, b3, 128)
    x4, _ = _conv_layer(x3, t3, W4f, W4d, g4, b4, 256)
    xc = jnp.concatenate([x1, x2, x3, x4], axis=-1).reshape(B, N, 3 * 512)
    w5 = jnp.concatenate(
        [_blockdiag(W5f.T, 3 * 512, 512, 1024),
         _blockdiag(W5d.T, 3 * 512, 512, 1024)], axis=1).astype(jnp.bfloat16)
    p5 = _mm5(xc, w5, 3 * 512, 1024)
    st5 = _stats5(p5, 1024)
    o5 = _apply5(st5, g5.reshape(1, 1024), b5.reshape(1, 1024), p5, 1024)
    return jnp.transpose(o5, (0, 2, 1))


# speedup vs baseline: 1.3819x; 1.1255x over previous
import functools

import jax
import jax.numpy as jnp
from jax import lax
from jax.experimental import pallas as pl
from jax.experimental.pallas import tpu as pltpu
from jax.experimental.pallas import tpu_sc as plsc

EPS = 1e-6
B = 2
N = 1024
K = 20
NBLK = 256
NBS = N // NBLK


def _knn_body(x_ref, idx_ref, idxg_ref, *, n, k):
    xb = x_ref[0]
    xbb = xb.astype(jnp.bfloat16)
    ip = lax.dot_general(xbb, xbb, (((0,), (0,)), ((), ())),
                         preferred_element_type=jnp.float32)
    inner = -2.0 * ip
    xx = jnp.sum(xb * xb, axis=0)
    pd = (-xx[None, :]) - inner - xx[:, None]
    iot = lax.broadcasted_iota(jnp.int32, (n, n), 1)
    off = pl.program_id(0) * n
    for t in range(k):
        m = jnp.max(pd, axis=1, keepdims=True)
        am = jnp.min(jnp.where(pd >= m, iot, n), axis=1).astype(jnp.int32)
        idx_ref[0, t, 0, :] = am
        idxg_ref[0, t, 0, :] = am + off
        pd = jnp.where(iot == am[:, None], -jnp.float32(3e38), pd)


def _knn(xt, cn):
    return pl.pallas_call(
        functools.partial(_knn_body, n=N, k=K),
        grid=(B,),
        in_specs=[pl.BlockSpec((1, cn, N), lambda b: (b, 0, 0))],
        out_specs=[pl.BlockSpec((1, K, 1, N), lambda b: (b, 0, 0, 0))] * 2,
        out_shape=[jax.ShapeDtypeStruct((B, K, 1, N), jnp.int32)] * 2,
    )(xt)


def _gather_body(idx_ref, x_ref, g_ref, *, n):
    idx = idx_ref[0, 0, 0, :]
    iot = lax.broadcasted_iota(jnp.int32, (n, idx.shape[0]), 0)
    oh = jnp.where(iot == idx[None, :], 1.0, 0.0).astype(jnp.float32)
    g_ref[0, 0] = lax.dot_general(oh, x_ref[0], (((0,), (0,)), ((), ())),
                                  preferred_element_type=jnp.float32,
                                  precision=lax.Precision.HIGHEST)


def _gather(idx, xr, crow):
    npb = NBLK
    return pl.pallas_call(
        functools.partial(_gather_body, n=N),
        grid=(B, K, N // npb),
        in_specs=[
            pl.BlockSpec((1, 1, 1, npb), lambda b, j, p: (b, j, 0, p)),
            pl.BlockSpec((1, N, crow), lambda b, j, p: (b, 0, 0)),
        ],
        out_specs=pl.BlockSpec((1, 1, npb, crow), lambda b, j, p: (b, j, p, 0)),
        out_shape=jax.ShapeDtypeStruct((B, K, N, crow), jnp.float32),
    )(idx, xr)


_SC_P = B * K * N
_SC_NW = 32
_SC_PERW = _SC_P // _SC_NW
_SC_CP = 128
_SC_CHUNKS = _SC_PERW // _SC_CP


def _gather_sc(idxg_flat, xrows, crow):
    mesh = plsc.VectorSubcoreMesh(core_axis_name="c", subcore_axis_name="s")

    @functools.partial(
        pl.kernel, mesh=mesh,
        out_type=jax.ShapeDtypeStruct((_SC_P, crow), jnp.float32),
        scratch_types=[
            pltpu.VMEM((_SC_CP,), jnp.int32),
            pltpu.VMEM((_SC_CP, crow), jnp.float32),
            pltpu.SemaphoreType.DMA,
        ],
    )
    def k(idx_hbm, tab_hbm, out_hbm, idx_v, rows_v, sem):
        wid = lax.axis_index("s") * 2 + lax.axis_index("c")
        for i in range(_SC_CHUNKS):
            base = wid * _SC_PERW + i * _SC_CP
            pltpu.sync_copy(idx_hbm.at[pl.ds(base, _SC_CP)], idx_v)
            pltpu.async_copy(tab_hbm.at[idx_v], rows_v, sem).wait()
            pltpu.sync_copy(rows_v, out_hbm.at[pl.ds(base, _SC_CP)])

    return k(idxg_flat, xrows)


def _pmm(g, xc, w):
    outs = []
    for d in range(3):
        gd = g[:, d, :]
        cd = xc[:, d, :]
        feat = jnp.concatenate([gd - cd, cd], axis=1).astype(jnp.bfloat16)
        outs.append(lax.dot_general(feat, w, (((1,), (0,)), ((), ())),
                                    preferred_element_type=jnp.float32))
    return outs


def _stats_body(g_ref, xc_ref, wf_ref, s_ref, *, cnt):
    ph = pl.program_id(0)
    first = ((pl.program_id(1) == 0) & (pl.program_id(2) == 0)
             & (pl.program_id(3) == 0))

    @pl.when((ph == 0) & first)
    def _():
        s_ref[...] = jnp.zeros_like(s_ref)

    p = _pmm(g_ref[0, 0], xc_ref[0], wf_ref[...])
    nsq = p[0] * p[0] + p[1] * p[1] + p[2] * p[2]
    norm = jnp.sqrt(nsq) + EPS

    @pl.when(ph == 0)
    def _():
        s_ref[0, :] += jnp.sum(norm, axis=0)

    @pl.when(ph == 1)
    def _():
        mean = s_ref[0, :] / cnt
        dev = norm - mean[None, :]
        s_ref[1, :] += jnp.sum(dev * dev, axis=0)


def _stats(g, xr4, wf, c8, cout):
    return pl.pallas_call(
        functools.partial(_stats_body, cnt=B * N * K),
        grid=(2, B, K, NBS),
        in_specs=[
            pl.BlockSpec((1, 1, NBLK, 3, c8),
                         lambda ph, b, j, nb: (b, j, nb, 0, 0)),
            pl.BlockSpec((1, NBLK, 3, c8), lambda ph, b, j, nb: (b, nb, 0, 0)),
            pl.BlockSpec((2 * c8, cout), lambda ph, b, j, nb: (0, 0)),
        ],
        out_specs=pl.BlockSpec((2, cout), lambda ph, b, j, nb: (0, 0)),
        out_shape=jax.ShapeDtypeStruct((2, cout), jnp.float32),
    )(g, xr4, wf)


def _apply_body(s_ref, gam_ref, bet_ref, g_ref, xc_ref, wf_ref, wd_ref, o_ref,
                *, cnt):
    j = pl.program_id(2)
    mean = s_ref[0, :] / cnt
    var = s_ref[1, :] / cnt
    istd = jnp.sqrt(var + 1e-5)
    gam = gam_ref[0]
    bet = bet_ref[0]

    g = g_ref[0, 0]
    xc = xc_ref[0]
    p = _pmm(g, xc, wf_ref[...])
    dd = _pmm(g, xc, wd_ref[...])
    nsq = p[0] * p[0] + p[1] * p[1] + p[2] * p[2]
    norm = jnp.sqrt(nsq) + EPS
    nbn = (norm - mean[None, :]) / istd[None, :] * gam[None, :] + bet[None, :]
    pr = [pi / norm * nbn for pi in p]
    dot = pr[0] * dd[0] + pr[1] * dd[1] + pr[2] * dd[2]
    dnsq = dd[0] * dd[0] + dd[1] * dd[1] + dd[2] * dd[2]
    coef = jnp.where(dot >= 0.0, 0.0, dot / (dnsq + EPS))

    @pl.when(j == 0)
    def _():
        o_ref[...] = jnp.zeros_like(o_ref)

    cout = gam_ref.shape[1]
    for d in range(3):
        od = pr[d] - coef * dd[d]
        o_ref[0, :, d, 0:cout] += od

    @pl.when(j == K - 1)
    def _():
        o_ref[...] *= (1.0 / K)


def _apply(stats, gam, bet, g, xr4, wf, wd, c8, cout, cpad):
    return pl.pallas_call(
        functools.partial(_apply_body, cnt=B * N * K),
        grid=(B, NBS, K),
        in_specs=[
            pl.BlockSpec((2, cout), lambda b, nb, j: (0, 0)),
            pl.BlockSpec((1, cout), lambda b, nb, j: (0, 0)),
            pl.BlockSpec((1, cout), lambda b, nb, j: (0, 0)),
            pl.BlockSpec((1, 1, NBLK, 3, c8), lambda b, nb, j: (b, j, nb, 0, 0)),
            pl.BlockSpec((1, NBLK, 3, c8), lambda b, nb, j: (b, nb, 0, 0)),
            pl.BlockSpec((2 * c8, cout), lambda b, nb, j: (0, 0)),
            pl.BlockSpec((2 * c8, cout), lambda b, nb, j: (0, 0)),
        ],
        out_specs=pl.BlockSpec((1, NBLK, 3, cpad), lambda b, nb, j: (b, nb, 0, 0)),
        out_shape=jax.ShapeDtypeStruct((B, N, 3, cpad), jnp.float32),
    )(stats, gam, bet, g, xr4, wf, wd)


def _mm5_body(x_ref, w_ref, p_ref):
    xb = x_ref[0].astype(jnp.bfloat16)
    p_ref[0] = lax.dot_general(xb, w_ref[...], (((1,), (0,)), ((), ())),
                               preferred_element_type=jnp.float32)


def _mm5(xr, w5, crow, segw):
    return pl.pallas_call(
        _mm5_body,
        grid=(B, 6),
        in_specs=[
            pl.BlockSpec((1, N, crow), lambda b, s: (b, 0, 0)),
            pl.BlockSpec((crow, segw), lambda b, s: (0, s)),
        ],
        out_specs=pl.BlockSpec((1, N, segw), lambda b, s: (b, 0, s)),
        out_shape=jax.ShapeDtypeStruct((B, N, 6 * segw), jnp.float32),
    )(xr, w5)


def _stats5_body(g_ref, s_ref, *, segw, cnt):
    ph = pl.program_id(0)
    first = (pl.program_id(1) == 0) & (pl.program_id(2) == 0)

    @pl.when((ph == 0) & first)
    def _():
        s_ref[...] = jnp.zeros_like(s_ref)

    g = g_ref[0]
    nsq = None
    for d in range(3):
        p = g[:, d * segw:(d + 1) * segw]
        nsq = p * p if nsq is None else nsq + p * p
    norm = jnp.sqrt(nsq) + EPS

    @pl.when(ph == 0)
    def _():
        s_ref[0, :] += jnp.sum(norm, axis=0)

    @pl.when(ph == 1)
    def _():
        mean = s_ref[0, :] / cnt
        dev = norm - mean[None, :]
        s_ref[1, :] += jnp.sum(dev * dev, axis=0)


def _stats5(p5, segw):
    return pl.pallas_call(
        functools.partial(_stats5_body, segw=segw, cnt=B * N),
        grid=(2, B, NBS),
        in_specs=[pl.BlockSpec((1, NBLK, 6 * segw),
                               lambda ph, b, nb: (b, nb, 0))],
        out_specs=pl.BlockSpec((2, segw), lambda ph, b, nb: (0, 0)),
        out_shape=jax.ShapeDtypeStruct((2, segw), jnp.float32),
    )(p5)


def _apply5_body(s_ref, gam_ref, bet_ref, g_ref, o_ref, *, segw, cnt):
    nb = pl.program_id(1)
    mean = s_ref[0, :] / cnt
    var = s_ref[1, :] / cnt
    istd = jnp.sqrt(var + 1e-5)
    gam = gam_ref[0]
    bet = bet_ref[0]

    g = g_ref[0]
    p = [g[:, d * segw:(d + 1) * segw] for d in range(3)]
    dd = [g[:, (3 + d) * segw:(4 + d) * segw] for d in range(3)]
    nsq = p[0] * p[0] + p[1] * p[1] + p[2] * p[2]
    norm = jnp.sqrt(nsq) + EPS
    nbn = (norm - mean[None, :]) / istd[None, :] * gam[None, :] + bet[None, :]
    pr = [pi / norm * nbn for pi in p]
    dot = pr[0] * dd[0] + pr[1] * dd[1] + pr[2] * dd[2]
    dnsq = dd[0] * dd[0] + dd[1] * dd[1] + dd[2] * dd[2]
    coef = jnp.where(dot >= 0.0, 0.0, dot / (dnsq + EPS))

    @pl.when(nb == 0)
    def _():
        o_ref[...] = jnp.zeros_like(o_ref)

    for d in range(3):
        od = pr[d] - coef * dd[d]
        o_ref[0, d, :] += jnp.sum(od, axis=0)

    @pl.when(nb == NBS - 1)
    def _():
        o_ref[...] *= (1.0 / N)


def _apply5(stats, gam, bet, p5, segw):
    return pl.pallas_call(
        functools.partial(_apply5_body, segw=segw, cnt=B * N),
        grid=(B, NBS),
        in_specs=[
            pl.BlockSpec((2, segw), lambda b, nb: (0, 0)),
            pl.BlockSpec((1, segw), lambda b, nb: (0, 0)),
            pl.BlockSpec((1, segw), lambda b, nb: (0, 0)),
            pl.BlockSpec((1, NBLK, 6 * segw), lambda b, nb: (b, nb, 0)),
        ],
        out_specs=pl.BlockSpec((1, 3, segw), lambda b, nb: (b, 0, 0)),
        out_shape=jax.ShapeDtypeStruct((B, 3, segw), jnp.float32),
    )(stats, gam, bet, p5)


def _blockdiag(wt, crow, c, segw):
    cout = wt.shape[1]
    m = jnp.zeros((crow, 3 * segw), dtype=wt.dtype)
    for d in range(3):
        m = m.at[d * c:(d + 1) * c, d * segw:d * segw + cout].set(wt)
    return m


def _prep_w2(w, c8):
    c = w.shape[1] // 2
    wa = jnp.pad(w[:, :c].T, ((0, c8 - c), (0, 0)))
    wb = jnp.pad(w[:, c:].T, ((0, c8 - c), (0, 0)))
    return jnp.concatenate([wa, wb], axis=0).astype(jnp.bfloat16)


def _conv_layer(xr4, xt, wf, wd, gam, bet, cout, cpad):
    c8 = xr4.shape[-1]
    crow = 3 * c8
    idx, idxg = _knn(xt, xt.shape[1])
    wfp = _prep_w2(wf, c8)
    wdp = _prep_w2(wd, c8)
    if crow % 128 == 0:
        g = _gather_sc(idxg.reshape(_SC_P), xr4.reshape(B * N, crow), crow)
        g = g.reshape(B, K, N, 3, c8)
    else:
        g = _gather(idx, xr4.reshape(B, N, crow), crow).reshape(B, K, N, 3, c8)
    stats = _stats(g, xr4, wfp, c8, cout)
    out = _apply(stats, gam.reshape(1, cout), bet.reshape(1, cout),
                 g, xr4, wfp, wdp, c8, cout, cpad)
    xt_next = jnp.transpose(out, (0, 3, 2, 1)).reshape(B, cpad * 3, N)[:, :cout * 3]
    return out, xt_next


def kernel(x, W1f, W1d, g1, b1, W2f, W2d, g2, b2, W3f, W3d, g3, b3,
           W4f, W4d, g4, b4, W5f, W5d, g5, b5):
    xr0 = jnp.pad(jnp.transpose(x, (0, 2, 1))[..., None],
                  ((0, 0), (0, 0), (0, 0), (0, 7)))
    x1, t1 = _conv_layer(xr0, x, W1f, W1d, g1, b1, 64, 128)
    x2, t2 = _conv_layer(x1, t1, W2f, W2d, g2, b2, 64, 128)
    x3, t3 = _conv_layer(x2, t2, W3f, W3d, g3, b3, 128, 128)
    x4, _ = _conv_layer(x3, t3, W4f, W4d, g4, b4, 256, 256)
    xc = jnp.concatenate([x1[..., :64], x2[..., :64], x3, x4],
                         axis=-1).reshape(B, N, 3 * 512)
    w5 = jnp.concatenate(
        [_blockdiag(W5f.T, 3 * 512, 512, 1024),
         _blockdiag(W5d.T, 3 * 512, 512, 1024)], axis=1).astype(jnp.bfloat16)
    p5 = _mm5(xc, w5, 3 * 512, 1024)
    st5 = _stats5(p5, 1024)
    o5 = _apply5(st5, g5.reshape(1, 1024), b5.reshape(1, 1024), p5, 1024)
    return jnp.transpose(o5, (0, 2, 1))


# speedup vs baseline: 1.8040x; 1.3054x over previous
import functools

import jax
import jax.numpy as jnp
from jax import lax
from jax.experimental import pallas as pl
from jax.experimental.pallas import tpu as pltpu
from jax.experimental.pallas import tpu_sc as plsc

EPS = 1e-6
B = 2
N = 1024
K = 20
NBLK = 256
NBS = N // NBLK


def _knn_body(x_ref, idx_ref, idxg_ref, *, n, k):
    xb = x_ref[0]
    xbb = xb.astype(jnp.bfloat16)
    ip = lax.dot_general(xbb, xbb, (((0,), (0,)), ((), ())),
                         preferred_element_type=jnp.float32)
    inner = -2.0 * ip
    xx = jnp.sum(xb * xb, axis=0)
    pd = (-xx[None, :]) - inner - xx[:, None]
    iot = lax.broadcasted_iota(jnp.int32, (n, n), 1)
    off = pl.program_id(0) * n
    for t in range(k):
        m = jnp.max(pd, axis=1, keepdims=True)
        am = jnp.min(jnp.where(pd >= m, iot, n), axis=1).astype(jnp.int32)
        idx_ref[0, t, 0, :] = am
        idxg_ref[0, t, 0, :] = am + off
        pd = jnp.where(iot == am[:, None], -jnp.float32(3e38), pd)


def _knn(xt, cn):
    return pl.pallas_call(
        functools.partial(_knn_body, n=N, k=K),
        grid=(B,),
        in_specs=[pl.BlockSpec((1, cn, N), lambda b: (b, 0, 0))],
        out_specs=[pl.BlockSpec((1, K, 1, N), lambda b: (b, 0, 0, 0))] * 2,
        out_shape=[jax.ShapeDtypeStruct((B, K, 1, N), jnp.int32)] * 2,
    )(xt)


def _gather_body(idx_ref, x_ref, g_ref, *, n):
    idx = idx_ref[0, 0, 0, :]
    iot = lax.broadcasted_iota(jnp.int32, (n, idx.shape[0]), 0)
    oh = jnp.where(iot == idx[None, :], 1.0, 0.0).astype(jnp.float32)
    g_ref[0, 0] = lax.dot_general(oh, x_ref[0], (((0,), (0,)), ((), ())),
                                  preferred_element_type=jnp.float32,
                                  precision=lax.Precision.HIGHEST)


def _gather(idx, xr, crow):
    npb = NBLK
    return pl.pallas_call(
        functools.partial(_gather_body, n=N),
        grid=(B, K, N // npb),
        in_specs=[
            pl.BlockSpec((1, 1, 1, npb), lambda b, j, p: (b, j, 0, p)),
            pl.BlockSpec((1, N, crow), lambda b, j, p: (b, 0, 0)),
        ],
        out_specs=pl.BlockSpec((1, 1, npb, crow), lambda b, j, p: (b, j, p, 0)),
        out_shape=jax.ShapeDtypeStruct((B, K, N, crow), jnp.float32),
    )(idx, xr)


_SC_P = B * K * N
_SC_NW = 32
_SC_PERW = _SC_P // _SC_NW
_SC_CP = 128
_SC_CHUNKS = _SC_PERW // _SC_CP


def _gather_sc(idxg_flat, xrows, crow):
    mesh = plsc.VectorSubcoreMesh(core_axis_name="c", subcore_axis_name="s")

    @functools.partial(
        pl.kernel, mesh=mesh,
        out_type=jax.ShapeDtypeStruct((_SC_P, crow), jnp.float32),
        scratch_types=[
            pltpu.VMEM((_SC_CP,), jnp.int32),
            pltpu.VMEM((_SC_CP, crow), jnp.float32),
            pltpu.SemaphoreType.DMA,
        ],
    )
    def k(idx_hbm, tab_hbm, out_hbm, idx_v, rows_v, sem):
        wid = lax.axis_index("s") * 2 + lax.axis_index("c")
        for i in range(_SC_CHUNKS):
            base = wid * _SC_PERW + i * _SC_CP
            pltpu.sync_copy(idx_hbm.at[pl.ds(base, _SC_CP)], idx_v)
            pltpu.async_copy(tab_hbm.at[idx_v], rows_v, sem).wait()
            pltpu.sync_copy(rows_v, out_hbm.at[pl.ds(base, _SC_CP)])

    return k(idxg_flat, xrows)


def _pmm(g, xc, w):
    outs = []
    for d in range(3):
        gd = g[:, d, :]
        cd = xc[:, d, :]
        feat = jnp.concatenate([gd - cd, cd], axis=1).astype(jnp.bfloat16)
        outs.append(lax.dot_general(feat, w, (((1,), (0,)), ((), ())),
                                    preferred_element_type=jnp.float32))
    return outs


_JB = 5


def _stats_body(g_ref, xc_ref, wf_ref, s_ref, *, cnt):
    ph = pl.program_id(0)
    first = (pl.program_id(1) == 0) & (pl.program_id(2) == 0)

    @pl.when((ph == 0) & first)
    def _():
        s_ref[...] = jnp.zeros_like(s_ref)

    xc = xc_ref[0]
    for jj in range(_JB):
        p = _pmm(g_ref[0, jj], xc, wf_ref[...])
        nsq = p[0] * p[0] + p[1] * p[1] + p[2] * p[2]
        norm = jnp.sqrt(nsq) + EPS

        @pl.when(ph == 0)
        def _():
            s_ref[0, :] += jnp.sum(norm, axis=0)

        @pl.when(ph == 1)
        def _():
            mean = s_ref[0, :] / cnt
            dev = norm - mean[None, :]
            s_ref[1, :] += jnp.sum(dev * dev, axis=0)


def _stats(g, xr4, wf, c8, cout):
    return pl.pallas_call(
        functools.partial(_stats_body, cnt=B * N * K),
        grid=(2, B, K // _JB),
        in_specs=[
            pl.BlockSpec((1, _JB, N, 3, c8),
                         lambda ph, b, jc: (b, jc, 0, 0, 0)),
            pl.BlockSpec((1, N, 3, c8), lambda ph, b, jc: (b, 0, 0, 0)),
            pl.BlockSpec((2 * c8, cout), lambda ph, b, jc: (0, 0)),
        ],
        out_specs=pl.BlockSpec((2, cout), lambda ph, b, jc: (0, 0)),
        out_shape=jax.ShapeDtypeStruct((2, cout), jnp.float32),
    )(g, xr4, wf)


def _apply_body(s_ref, gam_ref, bet_ref, g_ref, xc_ref, wf_ref, wd_ref, o_ref,
                *, cnt):
    jc = pl.program_id(1)
    mean = s_ref[0, :] / cnt
    var = s_ref[1, :] / cnt
    istd = jnp.sqrt(var + 1e-5)
    gam = gam_ref[0]
    bet = bet_ref[0]
    cout = gam_ref.shape[1]
    xc = xc_ref[0]

    @pl.when(jc == 0)
    def _():
        o_ref[...] = jnp.zeros_like(o_ref)

    for jj in range(_JB):
        p = _pmm(g_ref[0, jj], xc, wf_ref[...])
        dd = _pmm(g_ref[0, jj], xc, wd_ref[...])
        nsq = p[0] * p[0] + p[1] * p[1] + p[2] * p[2]
        norm = jnp.sqrt(nsq) + EPS
        nbn = (norm - mean[None, :]) / istd[None, :] * gam[None, :] \
            + bet[None, :]
        pr = [pi / norm * nbn for pi in p]
        dot = pr[0] * dd[0] + pr[1] * dd[1] + pr[2] * dd[2]
        dnsq = dd[0] * dd[0] + dd[1] * dd[1] + dd[2] * dd[2]
        coef = jnp.where(dot >= 0.0, 0.0, dot / (dnsq + EPS))
        for d in range(3):
            od = pr[d] - coef * dd[d]
            o_ref[0, :, d, 0:cout] += od

    @pl.when(jc == K // _JB - 1)
    def _():
        o_ref[...] *= (1.0 / K)


def _apply(stats, gam, bet, g, xr4, wf, wd, c8, cout, cpad):
    return pl.pallas_call(
        functools.partial(_apply_body, cnt=B * N * K),
        grid=(B, K // _JB),
        in_specs=[
            pl.BlockSpec((2, cout), lambda b, jc: (0, 0)),
            pl.BlockSpec((1, cout), lambda b, jc: (0, 0)),
            pl.BlockSpec((1, cout), lambda b, jc: (0, 0)),
            pl.BlockSpec((1, _JB, N, 3, c8), lambda b, jc: (b, jc, 0, 0, 0)),
            pl.BlockSpec((1, N, 3, c8), lambda b, jc: (b, 0, 0, 0)),
            pl.BlockSpec((2 * c8, cout), lambda b, jc: (0, 0)),
            pl.BlockSpec((2 * c8, cout), lambda b, jc: (0, 0)),
        ],
        out_specs=pl.BlockSpec((1, N, 3, cpad), lambda b, jc: (b, 0, 0, 0)),
        out_shape=jax.ShapeDtypeStruct((B, N, 3, cpad), jnp.float32),
    )(stats, gam, bet, g, xr4, wf, wd)


def _mm5_body(x_ref, w_ref, p_ref):
    xb = x_ref[0].astype(jnp.bfloat16)
    p_ref[0] = lax.dot_general(xb, w_ref[...], (((1,), (0,)), ((), ())),
                               preferred_element_type=jnp.float32)


def _mm5(xr, w5, crow, segw):
    return pl.pallas_call(
        _mm5_body,
        grid=(B, 6),
        in_specs=[
            pl.BlockSpec((1, N, crow), lambda b, s: (b, 0, 0)),
            pl.BlockSpec((crow, segw), lambda b, s: (0, s)),
        ],
        out_specs=pl.BlockSpec((1, N, segw), lambda b, s: (b, 0, s)),
        out_shape=jax.ShapeDtypeStruct((B, N, 6 * segw), jnp.float32),
    )(xr, w5)


def _stats5_body(g_ref, s_ref, *, segw, cnt):
    ph = pl.program_id(0)
    first = (pl.program_id(1) == 0) & (pl.program_id(2) == 0)

    @pl.when((ph == 0) & first)
    def _():
        s_ref[...] = jnp.zeros_like(s_ref)

    g = g_ref[0]
    nsq = None
    for d in range(3):
        p = g[:, d * segw:(d + 1) * segw]
        nsq = p * p if nsq is None else nsq + p * p
    norm = jnp.sqrt(nsq) + EPS

    @pl.when(ph == 0)
    def _():
        s_ref[0, :] += jnp.sum(norm, axis=0)

    @pl.when(ph == 1)
    def _():
        mean = s_ref[0, :] / cnt
        dev = norm - mean[None, :]
        s_ref[1, :] += jnp.sum(dev * dev, axis=0)


def _stats5(p5, segw):
    return pl.pallas_call(
        functools.partial(_stats5_body, segw=segw, cnt=B * N),
        grid=(2, B, NBS),
        in_specs=[pl.BlockSpec((1, NBLK, 6 * segw),
                               lambda ph, b, nb: (b, nb, 0))],
        out_specs=pl.BlockSpec((2, segw), lambda ph, b, nb: (0, 0)),
        out_shape=jax.ShapeDtypeStruct((2, segw), jnp.float32),
    )(p5)


def _apply5_body(s_ref, gam_ref, bet_ref, g_ref, o_ref, *, segw, cnt):
    nb = pl.program_id(1)
    mean = s_ref[0, :] / cnt
    var = s_ref[1, :] / cnt
    istd = jnp.sqrt(var + 1e-5)
    gam = gam_ref[0]
    bet = bet_ref[0]

    g = g_ref[0]
    p = [g[:, d * segw:(d + 1) * segw] for d in range(3)]
    dd = [g[:, (3 + d) * segw:(4 + d) * segw] for d in range(3)]
    nsq = p[0] * p[0] + p[1] * p[1] + p[2] * p[2]
    norm = jnp.sqrt(nsq) + EPS
    nbn = (norm - mean[None, :]) / istd[None, :] * gam[None, :] + bet[None, :]
    pr = [pi / norm * nbn for pi in p]
    dot = pr[0] * dd[0] + pr[1] * dd[1] + pr[2] * dd[2]
    dnsq = dd[0] * dd[0] + dd[1] * dd[1] + dd[2] * dd[2]
    coef = jnp.where(dot >= 0.0, 0.0, dot / (dnsq + EPS))

    @pl.when(nb == 0)
    def _():
        o_ref[...] = jnp.zeros_like(o_ref)

    for d in range(3):
        od = pr[d] - coef * dd[d]
        o_ref[0, d, :] += jnp.sum(od, axis=0)

    @pl.when(nb == NBS - 1)
    def _():
        o_ref[...] *= (1.0 / N)


def _apply5(stats, gam, bet, p5, segw):
    return pl.pallas_call(
        functools.partial(_apply5_body, segw=segw, cnt=B * N),
        grid=(B, NBS),
        in_specs=[
            pl.BlockSpec((2, segw), lambda b, nb: (0, 0)),
            pl.BlockSpec((1, segw), lambda b, nb: (0, 0)),
            pl.BlockSpec((1, segw), lambda b, nb: (0, 0)),
            pl.BlockSpec((1, NBLK, 6 * segw), lambda b, nb: (b, nb, 0)),
        ],
        out_specs=pl.BlockSpec((1, 3, segw), lambda b, nb: (b, 0, 0)),
        out_shape=jax.ShapeDtypeStruct((B, 3, segw), jnp.float32),
    )(stats, gam, bet, p5)


def _blockdiag(wt, crow, c, segw):
    cout = wt.shape[1]
    m = jnp.zeros((crow, 3 * segw), dtype=wt.dtype)
    for d in range(3):
        m = m.at[d * c:(d + 1) * c, d * segw:d * segw + cout].set(wt)
    return m


def _prep_w2(w, c8):
    c = w.shape[1] // 2
    wa = jnp.pad(w[:, :c].T, ((0, c8 - c), (0, 0)))
    wb = jnp.pad(w[:, c:].T, ((0, c8 - c), (0, 0)))
    return jnp.concatenate([wa, wb], axis=0).astype(jnp.bfloat16)


def _conv_layer(xr4, xt, wf, wd, gam, bet, cout, cpad):
    c8 = xr4.shape[-1]
    crow = 3 * c8
    idx, idxg = _knn(xt, xt.shape[1])
    wfp = _prep_w2(wf, c8)
    wdp = _prep_w2(wd, c8)
    if crow % 128 == 0:
        g = _gather_sc(idxg.reshape(_SC_P), xr4.reshape(B * N, crow), crow)
        g = g.reshape(B, K, N, 3, c8)
    else:
        g = _gather(idx, xr4.reshape(B, N, crow), crow).reshape(B, K, N, 3, c8)
    stats = _stats(g, xr4, wfp, c8, cout)
    out = _apply(stats, gam.reshape(1, cout), bet.reshape(1, cout),
                 g, xr4, wfp, wdp, c8, cout, cpad)
    xt_next = jnp.transpose(out, (0, 3, 2, 1)).reshape(B, cpad * 3, N)[:, :cout * 3]
    return out, xt_next


def kernel(x, W1f, W1d, g1, b1, W2f, W2d, g2, b2, W3f, W3d, g3, b3,
           W4f, W4d, g4, b4, W5f, W5d, g5, b5):
    xr0 = jnp.pad(jnp.transpose(x, (0, 2, 1))[..., None],
                  ((0, 0), (0, 0), (0, 0), (0, 7)))
    x1, t1 = _conv_layer(xr0, x, W1f, W1d, g1, b1, 64, 128)
    x2, t2 = _conv_layer(x1, t1, W2f, W2d, g2, b2, 64, 128)
    x3, t3 = _conv_layer(x2, t2, W3f, W3d, g3, b3, 128, 128)
    x4, _ = _conv_layer(x3, t3, W4f, W4d, g4, b4, 256, 256)
    xc = jnp.concatenate([x1[..., :64], x2[..., :64], x3, x4],
                         axis=-1).reshape(B, N, 3 * 512)
    w5 = jnp.concatenate(
        [_blockdiag(W5f.T, 3 * 512, 512, 1024),
         _blockdiag(W5d.T, 3 * 512, 512, 1024)], axis=1).astype(jnp.bfloat16)
    p5 = _mm5(xc, w5, 3 * 512, 1024)
    st5 = _stats5(p5, 1024)
    o5 = _apply5(st5, g5.reshape(1, 1024), b5.reshape(1, 1024), p5, 1024)
    return jnp.transpose(o5, (0, 2, 1))


# speedup vs baseline: 1.8321x; 1.0156x over previous
import functools

import jax
import jax.numpy as jnp
from jax import lax
from jax.experimental import pallas as pl
from jax.experimental.pallas import tpu as pltpu
from jax.experimental.pallas import tpu_sc as plsc

EPS = 1e-6
B = 2
N = 1024
K = 20
NBLK = 256
NBS = N // NBLK


def _knn_body(x_ref, idx_ref, idxg_ref, *, n, k):
    xb = x_ref[0]
    xbb = xb.astype(jnp.bfloat16)
    ip = lax.dot_general(xbb, xbb, (((0,), (0,)), ((), ())),
                         preferred_element_type=jnp.float32)
    inner = -2.0 * ip
    xx = jnp.sum(xb * xb, axis=0)
    pd = (-xx[None, :]) - inner - xx[:, None]
    iot = lax.broadcasted_iota(jnp.int32, (n, n), 1)
    off = pl.program_id(0) * n
    for t in range(k):
        am = jnp.argmax(pd, axis=1).astype(jnp.int32)
        idx_ref[0, t, 0, :] = am
        idxg_ref[0, t, 0, :] = am + off
        pd = jnp.where(iot == am[:, None], -jnp.float32(3e38), pd)


def _knn(xt, cn):
    return pl.pallas_call(
        functools.partial(_knn_body, n=N, k=K),
        grid=(B,),
        in_specs=[pl.BlockSpec((1, cn, N), lambda b: (b, 0, 0))],
        out_specs=[pl.BlockSpec((1, K, 1, N), lambda b: (b, 0, 0, 0))] * 2,
        out_shape=[jax.ShapeDtypeStruct((B, K, 1, N), jnp.int32)] * 2,
    )(xt)


def _gather_body(idx_ref, x_ref, g_ref, *, n):
    idx = idx_ref[0, 0, 0, :]
    iot = lax.broadcasted_iota(jnp.int32, (n, idx.shape[0]), 0)
    oh = jnp.where(iot == idx[None, :], 1.0, 0.0).astype(jnp.float32)
    g_ref[0, 0] = lax.dot_general(oh, x_ref[0], (((0,), (0,)), ((), ())),
                                  preferred_element_type=jnp.float32,
                                  precision=lax.Precision.HIGHEST)


def _gather(idx, xr, crow):
    return pl.pallas_call(
        functools.partial(_gather_body, n=N),
        grid=(B, K),
        in_specs=[
            pl.BlockSpec((1, 1, 1, N), lambda b, j: (b, j, 0, 0)),
            pl.BlockSpec((1, N, crow), lambda b, j: (b, 0, 0)),
        ],
        out_specs=pl.BlockSpec((1, 1, N, crow), lambda b, j: (b, j, 0, 0)),
        out_shape=jax.ShapeDtypeStruct((B, K, N, crow), jnp.float32),
    )(idx, xr)


_SC_P = B * K * N
_SC_NW = 32
_SC_PERW = _SC_P // _SC_NW
_SC_CP = 128
_SC_CHUNKS = _SC_PERW // _SC_CP


def _gather_sc(idxg_flat, xrows, crow):
    mesh = plsc.VectorSubcoreMesh(core_axis_name="c", subcore_axis_name="s")

    @functools.partial(
        pl.kernel, mesh=mesh,
        out_type=jax.ShapeDtypeStruct((_SC_P, crow), jnp.float32),
        scratch_types=[
            pltpu.VMEM((_SC_CP,), jnp.int32),
            pltpu.VMEM((_SC_CP, crow), jnp.float32),
            pltpu.SemaphoreType.DMA,
        ],
    )
    def k(idx_hbm, tab_hbm, out_hbm, idx_v, rows_v, sem):
        wid = lax.axis_index("s") * 2 + lax.axis_index("c")
        for i in range(_SC_CHUNKS):
            base = wid * _SC_PERW + i * _SC_CP
            pltpu.sync_copy(idx_hbm.at[pl.ds(base, _SC_CP)], idx_v)
            pltpu.async_copy(tab_hbm.at[idx_v], rows_v, sem).wait()
            pltpu.sync_copy(rows_v, out_hbm.at[pl.ds(base, _SC_CP)])

    return k(idxg_flat, xrows)


def _pmm(g, xc, w):
    outs = []
    for d in range(3):
        gd = g[:, d, :]
        cd = xc[:, d, :]
        feat = jnp.concatenate([gd - cd, cd], axis=1).astype(jnp.bfloat16)
        outs.append(lax.dot_general(feat, w, (((1,), (0,)), ((), ())),
                                    preferred_element_type=jnp.float32))
    return outs


_JB = 5


def _stats_body(g_ref, xc_ref, wf_ref, s_ref, *, cnt):
    ph = pl.program_id(0)
    first = (pl.program_id(1) == 0) & (pl.program_id(2) == 0)

    @pl.when((ph == 0) & first)
    def _():
        s_ref[...] = jnp.zeros_like(s_ref)

    xc = xc_ref[0]
    for jj in range(_JB):
        p = _pmm(g_ref[0, jj], xc, wf_ref[...])
        nsq = p[0] * p[0] + p[1] * p[1] + p[2] * p[2]
        norm = jnp.sqrt(nsq) + EPS

        @pl.when(ph == 0)
        def _():
            s_ref[0, :] += jnp.sum(norm, axis=0)

        @pl.when(ph == 1)
        def _():
            mean = s_ref[0, :] / cnt
            dev = norm - mean[None, :]
            s_ref[1, :] += jnp.sum(dev * dev, axis=0)


def _stats(g, xr4, wf, c8, cout):
    return pl.pallas_call(
        functools.partial(_stats_body, cnt=B * N * K),
        grid=(2, B, K // _JB),
        in_specs=[
            pl.BlockSpec((1, _JB, N, 3, c8),
                         lambda ph, b, jc: (b, jc, 0, 0, 0)),
            pl.BlockSpec((1, N, 3, c8), lambda ph, b, jc: (b, 0, 0, 0)),
            pl.BlockSpec((2 * c8, cout), lambda ph, b, jc: (0, 0)),
        ],
        out_specs=pl.BlockSpec((2, cout), lambda ph, b, jc: (0, 0)),
        out_shape=jax.ShapeDtypeStruct((2, cout), jnp.float32),
    )(g, xr4, wf)


def _apply_body(s_ref, gam_ref, bet_ref, g_ref, xc_ref, wf_ref, wd_ref, o_ref,
                *, cnt):
    jc = pl.program_id(1)
    mean = s_ref[0, :] / cnt
    var = s_ref[1, :] / cnt
    istd = jnp.sqrt(var + 1e-5)
    gam = gam_ref[0]
    bet = bet_ref[0]
    cout = gam_ref.shape[1]
    xc = xc_ref[0]

    @pl.when(jc == 0)
    def _():
        o_ref[...] = jnp.zeros_like(o_ref)

    for jj in range(_JB):
        p = _pmm(g_ref[0, jj], xc, wf_ref[...])
        dd = _pmm(g_ref[0, jj], xc, wd_ref[...])
        nsq = p[0] * p[0] + p[1] * p[1] + p[2] * p[2]
        norm = jnp.sqrt(nsq) + EPS
        nbn = (norm - mean[None, :]) / istd[None, :] * gam[None, :] \
            + bet[None, :]
        pr = [pi / norm * nbn for pi in p]
        dot = pr[0] * dd[0] + pr[1] * dd[1] + pr[2] * dd[2]
        dnsq = dd[0] * dd[0] + dd[1] * dd[1] + dd[2] * dd[2]
        coef = jnp.where(dot >= 0.0, 0.0, dot / (dnsq + EPS))
        for d in range(3):
            od = pr[d] - coef * dd[d]
            o_ref[0, :, d, 0:cout] += od

    @pl.when(jc == K // _JB - 1)
    def _():
        o_ref[...] *= (1.0 / K)


def _apply(stats, gam, bet, g, xr4, wf, wd, c8, cout, cpad):
    return pl.pallas_call(
        functools.partial(_apply_body, cnt=B * N * K),
        grid=(B, K // _JB),
        in_specs=[
            pl.BlockSpec((2, cout), lambda b, jc: (0, 0)),
            pl.BlockSpec((1, cout), lambda b, jc: (0, 0)),
            pl.BlockSpec((1, cout), lambda b, jc: (0, 0)),
            pl.BlockSpec((1, _JB, N, 3, c8), lambda b, jc: (b, jc, 0, 0, 0)),
            pl.BlockSpec((1, N, 3, c8), lambda b, jc: (b, 0, 0, 0)),
            pl.BlockSpec((2 * c8, cout), lambda b, jc: (0, 0)),
            pl.BlockSpec((2 * c8, cout), lambda b, jc: (0, 0)),
        ],
        out_specs=pl.BlockSpec((1, N, 3, cpad), lambda b, jc: (b, 0, 0, 0)),
        out_shape=jax.ShapeDtypeStruct((B, N, 3, cpad), jnp.float32),
    )(stats, gam, bet, g, xr4, wf, wd)


def _mm5_body(x_ref, w_ref, p_ref):
    xb = x_ref[0].astype(jnp.bfloat16)
    p_ref[0] = lax.dot_general(xb, w_ref[...], (((1,), (0,)), ((), ())),
                               preferred_element_type=jnp.float32)


def _mm5(xr, w5, crow, segw):
    return pl.pallas_call(
        _mm5_body,
        grid=(B, 6),
        in_specs=[
            pl.BlockSpec((1, N, crow), lambda b, s: (b, 0, 0)),
            pl.BlockSpec((crow, segw), lambda b, s: (0, s)),
        ],
        out_specs=pl.BlockSpec((1, N, segw), lambda b, s: (b, 0, s)),
        out_shape=jax.ShapeDtypeStruct((B, N, 6 * segw), jnp.float32),
    )(xr, w5)


def _stats5_body(g_ref, s_ref, *, segw, cnt):
    ph = pl.program_id(0)
    first = (pl.program_id(1) == 0) & (pl.program_id(2) == 0)

    @pl.when((ph == 0) & first)
    def _():
        s_ref[...] = jnp.zeros_like(s_ref)

    g = g_ref[0]
    nsq = None
    for d in range(3):
        p = g[:, d * segw:(d + 1) * segw]
        nsq = p * p if nsq is None else nsq + p * p
    norm = jnp.sqrt(nsq) + EPS

    @pl.when(ph == 0)
    def _():
        s_ref[0, :] += jnp.sum(norm, axis=0)

    @pl.when(ph == 1)
    def _():
        mean = s_ref[0, :] / cnt
        dev = norm - mean[None, :]
        s_ref[1, :] += jnp.sum(dev * dev, axis=0)


def _stats5(p5, segw):
    return pl.pallas_call(
        functools.partial(_stats5_body, segw=segw, cnt=B * N),
        grid=(2, B, N // 512),
        in_specs=[pl.BlockSpec((1, 512, 6 * segw),
                               lambda ph, b, nb: (b, nb, 0))],
        out_specs=pl.BlockSpec((2, segw), lambda ph, b, nb: (0, 0)),
        out_shape=jax.ShapeDtypeStruct((2, segw), jnp.float32),
    )(p5)


def _apply5_body(s_ref, gam_ref, bet_ref, g_ref, o_ref, *, segw, cnt):
    nb = pl.program_id(1)
    mean = s_ref[0, :] / cnt
    var = s_ref[1, :] / cnt
    istd = jnp.sqrt(var + 1e-5)
    gam = gam_ref[0]
    bet = bet_ref[0]

    g = g_ref[0]
    p = [g[:, d * segw:(d + 1) * segw] for d in range(3)]
    dd = [g[:, (3 + d) * segw:(4 + d) * segw] for d in range(3)]
    nsq = p[0] * p[0] + p[1] * p[1] + p[2] * p[2]
    norm = jnp.sqrt(nsq) + EPS
    nbn = (norm - mean[None, :]) / istd[None, :] * gam[None, :] + bet[None, :]
    pr = [pi / norm * nbn for pi in p]
    dot = pr[0] * dd[0] + pr[1] * dd[1] + pr[2] * dd[2]
    dnsq = dd[0] * dd[0] + dd[1] * dd[1] + dd[2] * dd[2]
    coef = jnp.where(dot >= 0.0, 0.0, dot / (dnsq + EPS))

    @pl.when(nb == 0)
    def _():
        o_ref[...] = jnp.zeros_like(o_ref)

    for d in range(3):
        od = pr[d] - coef * dd[d]
        o_ref[0, d, :] += jnp.sum(od, axis=0)

    @pl.when(nb == N // 512 - 1)
    def _():
        o_ref[...] *= (1.0 / N)


def _apply5(stats, gam, bet, p5, segw):
    return pl.pallas_call(
        functools.partial(_apply5_body, segw=segw, cnt=B * N),
        grid=(B, N // 512),
        in_specs=[
            pl.BlockSpec((2, segw), lambda b, nb: (0, 0)),
            pl.BlockSpec((1, segw), lambda b, nb: (0, 0)),
            pl.BlockSpec((1, segw), lambda b, nb: (0, 0)),
            pl.BlockSpec((1, 512, 6 * segw), lambda b, nb: (b, nb, 0)),
        ],
        out_specs=pl.BlockSpec((1, 3, segw), lambda b, nb: (b, 0, 0)),
        out_shape=jax.ShapeDtypeStruct((B, 3, segw), jnp.float32),
    )(stats, gam, bet, p5)


def _blockdiag(wt, crow, c, segw):
    cout = wt.shape[1]
    m = jnp.zeros((crow, 3 * segw), dtype=wt.dtype)
    for d in range(3):
        m = m.at[d * c:(d + 1) * c, d * segw:d * segw + cout].set(wt)
    return m


def _prep_w2(w, c8):
    c = w.shape[1] // 2
    wa = jnp.pad(w[:, :c].T, ((0, c8 - c), (0, 0)))
    wb = jnp.pad(w[:, c:].T, ((0, c8 - c), (0, 0)))
    return jnp.concatenate([wa, wb], axis=0).astype(jnp.bfloat16)


def _conv_layer(xr4, xt, wf, wd, gam, bet, cout, cpad):
    c8 = xr4.shape[-1]
    crow = 3 * c8
    idx, idxg = _knn(xt, xt.shape[1])
    wfp = _prep_w2(wf, c8)
    wdp = _prep_w2(wd, c8)
    if crow % 128 == 0:
        g = _gather_sc(idxg.reshape(_SC_P), xr4.reshape(B * N, crow), crow)
        g = g.reshape(B, K, N, 3, c8)
    else:
        g = _gather(idx, xr4.reshape(B, N, crow), crow).reshape(B, K, N, 3, c8)
    stats = _stats(g, xr4, wfp, c8, cout)
    out = _apply(stats, gam.reshape(1, cout), bet.reshape(1, cout),
                 g, xr4, wfp, wdp, c8, cout, cpad)
    xt_next = jnp.transpose(out, (0, 3, 2, 1)).reshape(B, cpad * 3, N)[:, :cout * 3]
    return out, xt_next


def kernel(x, W1f, W1d, g1, b1, W2f, W2d, g2, b2, W3f, W3d, g3, b3,
           W4f, W4d, g4, b4, W5f, W5d, g5, b5):
    xr0 = jnp.pad(jnp.transpose(x, (0, 2, 1))[..., None],
                  ((0, 0), (0, 0), (0, 0), (0, 7)))
    x1, t1 = _conv_layer(xr0, x, W1f, W1d, g1, b1, 64, 128)
    x2, t2 = _conv_layer(x1, t1, W2f, W2d, g2, b2, 64, 128)
    x3, t3 = _conv_layer(x2, t2, W3f, W3d, g3, b3, 128, 128)
    x4, _ = _conv_layer(x3, t3, W4f, W4d, g4, b4, 256, 256)
    xc = jnp.concatenate([x1[..., :64], x2[..., :64], x3, x4],
                         axis=-1).reshape(B, N, 3 * 512)
    w5 = jnp.concatenate(
        [_blockdiag(W5f.T, 3 * 512, 512, 1024),
         _blockdiag(W5d.T, 3 * 512, 512, 1024)], axis=1).astype(jnp.bfloat16)
    p5 = _mm5(xc, w5, 3 * 512, 1024)
    st5 = _stats5(p5, 1024)
    o5 = _apply5(st5, g5.reshape(1, 1024), b5.reshape(1, 1024), p5, 1024)
    return jnp.transpose(o5, (0, 2, 1))


# speedup vs baseline: 1.9831x; 1.0824x over previous
import functools

import jax
import jax.numpy as jnp
from jax import lax
from jax.experimental import pallas as pl
from jax.experimental.pallas import tpu as pltpu
from jax.experimental.pallas import tpu_sc as plsc

EPS = 1e-6
B = 2
N = 1024
K = 20
NBLK = 256
NBS = N // NBLK


def _knn_body(x_ref, idx_ref, idxg_ref, *, n, k):
    xb = x_ref[0]
    xbb = xb.astype(jnp.bfloat16)
    ip = lax.dot_general(xbb, xbb, (((0,), (0,)), ((), ())),
                         preferred_element_type=jnp.float32)
    inner = -2.0 * ip
    xx = jnp.sum(xb * xb, axis=0)
    pd = (-xx[None, :]) - inner - xx[:, None]
    iot = lax.broadcasted_iota(jnp.int32, (n, n), 1)
    off = pl.program_id(0) * n
    for t in range(k):
        am = jnp.argmax(pd, axis=1).astype(jnp.int32)
        idx_ref[0, t, 0, :] = am
        idxg_ref[0, t, 0, :] = am + off
        pd = jnp.where(iot == am[:, None], -jnp.float32(3e38), pd)


def _knn(xt, cn):
    return pl.pallas_call(
        functools.partial(_knn_body, n=N, k=K),
        grid=(B,),
        in_specs=[pl.BlockSpec((1, cn, N), lambda b: (b, 0, 0))],
        out_specs=[pl.BlockSpec((1, K, 1, N), lambda b: (b, 0, 0, 0))] * 2,
        out_shape=[jax.ShapeDtypeStruct((B, K, 1, N), jnp.int32)] * 2,
    )(xt)


def _gather_body(idx_ref, x_ref, g_ref, *, n):
    idx = idx_ref[0, 0, 0, :]
    iot = lax.broadcasted_iota(jnp.int32, (n, idx.shape[0]), 0)
    oh = jnp.where(iot == idx[None, :], 1.0, 0.0).astype(jnp.float32)
    g_ref[0, 0] = lax.dot_general(oh, x_ref[0], (((0,), (0,)), ((), ())),
                                  preferred_element_type=jnp.float32,
                                  precision=lax.Precision.HIGHEST)


def _gather(idx, xr, crow):
    return pl.pallas_call(
        functools.partial(_gather_body, n=N),
        grid=(B, K),
        in_specs=[
            pl.BlockSpec((1, 1, 1, N), lambda b, j: (b, j, 0, 0)),
            pl.BlockSpec((1, N, crow), lambda b, j: (b, 0, 0)),
        ],
        out_specs=pl.BlockSpec((1, 1, N, crow), lambda b, j: (b, j, 0, 0)),
        out_shape=jax.ShapeDtypeStruct((B, K, N, crow), jnp.float32),
    )(idx, xr)


_SC_P = B * K * N
_SC_NW = 32
_SC_PERW = _SC_P // _SC_NW
_SC_CP = 128
_SC_CHUNKS = _SC_PERW // _SC_CP


def _gather_sc(idxg_flat, xrows, crow):
    mesh = plsc.VectorSubcoreMesh(core_axis_name="c", subcore_axis_name="s")

    @functools.partial(
        pl.kernel, mesh=mesh,
        out_type=jax.ShapeDtypeStruct((_SC_P, crow), jnp.float32),
        scratch_types=[
            pltpu.VMEM((_SC_CP,), jnp.int32),
            pltpu.VMEM((_SC_CP, crow), jnp.float32),
            pltpu.SemaphoreType.DMA,
        ],
    )
    def k(idx_hbm, tab_hbm, out_hbm, idx_v, rows_v, sem):
        wid = lax.axis_index("s") * 2 + lax.axis_index("c")
        for i in range(_SC_CHUNKS):
            base = wid * _SC_PERW + i * _SC_CP
            pltpu.sync_copy(idx_hbm.at[pl.ds(base, _SC_CP)], idx_v)
            pltpu.async_copy(tab_hbm.at[idx_v], rows_v, sem).wait()
            pltpu.sync_copy(rows_v, out_hbm.at[pl.ds(base, _SC_CP)])

    return k(idxg_flat, xrows)


def _pmm(g, xc, w):
    outs = []
    for d in range(3):
        gd = g[:, d, :]
        cd = xc[:, d, :]
        feat = jnp.concatenate([gd - cd, cd], axis=1).astype(jnp.bfloat16)
        outs.append(lax.dot_general(feat, w, (((1,), (0,)), ((), ())),
                                    preferred_element_type=jnp.float32))
    return outs


_JB = 5


def _stats0_body(g_ref, xc_ref, wf_ref, s_ref, nrm_ref):
    first = (pl.program_id(0) == 0) & (pl.program_id(1) == 0)

    @pl.when(first)
    def _():
        s_ref[...] = jnp.zeros_like(s_ref)

    xc = xc_ref[0]
    for jj in range(_JB):
        p = _pmm(g_ref[0, jj], xc, wf_ref[...])
        nsq = p[0] * p[0] + p[1] * p[1] + p[2] * p[2]
        norm = jnp.sqrt(nsq) + EPS
        s_ref[0, :] += jnp.sum(norm, axis=0)
        nrm_ref[0, jj] = norm


def _stats1_body(nrm_ref, s0_ref, v_ref, *, cnt):
    first = (pl.program_id(0) == 0) & (pl.program_id(1) == 0)

    @pl.when(first)
    def _():
        v_ref[...] = jnp.zeros_like(v_ref)

    mean = s0_ref[0, :] / cnt
    for jj in range(_JB):
        dev = nrm_ref[0, jj] - mean[None, :]
        v_ref[0, :] += jnp.sum(dev * dev, axis=0)


def _stats(g, xr4, wf, c8, cout):
    s0, nrm = pl.pallas_call(
        _stats0_body,
        grid=(B, K // _JB),
        in_specs=[
            pl.BlockSpec((1, _JB, N, 3, c8), lambda b, jc: (b, jc, 0, 0, 0)),
            pl.BlockSpec((1, N, 3, c8), lambda b, jc: (b, 0, 0, 0)),
            pl.BlockSpec((2 * c8, cout), lambda b, jc: (0, 0)),
        ],
        out_specs=[
            pl.BlockSpec((1, cout), lambda b, jc: (0, 0)),
            pl.BlockSpec((1, _JB, N, cout), lambda b, jc: (b, jc, 0, 0)),
        ],
        out_shape=[
            jax.ShapeDtypeStruct((1, cout), jnp.float32),
            jax.ShapeDtypeStruct((B, K, N, cout), jnp.float32),
        ],
    )(g, xr4, wf)
    v = pl.pallas_call(
        functools.partial(_stats1_body, cnt=B * N * K),
        grid=(B, K // _JB),
        in_specs=[
            pl.BlockSpec((1, _JB, N, cout), lambda b, jc: (b, jc, 0, 0)),
            pl.BlockSpec((1, cout), lambda b, jc: (0, 0)),
        ],
        out_specs=pl.BlockSpec((1, cout), lambda b, jc: (0, 0)),
        out_shape=jax.ShapeDtypeStruct((1, cout), jnp.float32),
    )(nrm, s0)
    return jnp.concatenate([s0, v], axis=0)


def _apply_body(s_ref, gam_ref, bet_ref, g_ref, xc_ref, wf_ref, wd_ref, o_ref,
                *, cnt):
    jc = pl.program_id(1)
    mean = s_ref[0, :] / cnt
    var = s_ref[1, :] / cnt
    istd = jnp.sqrt(var + 1e-5)
    gam = gam_ref[0]
    bet = bet_ref[0]
    cout = gam_ref.shape[1]
    xc = xc_ref[0]

    @pl.when(jc == 0)
    def _():
        o_ref[...] = jnp.zeros_like(o_ref)

    for jj in range(_JB):
        p = _pmm(g_ref[0, jj], xc, wf_ref[...])
        dd = _pmm(g_ref[0, jj], xc, wd_ref[...])
        nsq = p[0] * p[0] + p[1] * p[1] + p[2] * p[2]
        norm = jnp.sqrt(nsq) + EPS
        nbn = (norm - mean[None, :]) / istd[None, :] * gam[None, :] \
            + bet[None, :]
        pr = [pi / norm * nbn for pi in p]
        dot = pr[0] * dd[0] + pr[1] * dd[1] + pr[2] * dd[2]
        dnsq = dd[0] * dd[0] + dd[1] * dd[1] + dd[2] * dd[2]
        coef = jnp.where(dot >= 0.0, 0.0, dot / (dnsq + EPS))
        for d in range(3):
            od = pr[d] - coef * dd[d]
            o_ref[0, :, d, 0:cout] += od

    @pl.when(jc == K // _JB - 1)
    def _():
        o_ref[...] *= (1.0 / K)


def _apply(stats, gam, bet, g, xr4, wf, wd, c8, cout, cpad):
    return pl.pallas_call(
        functools.partial(_apply_body, cnt=B * N * K),
        grid=(B, K // _JB),
        in_specs=[
            pl.BlockSpec((2, cout), lambda b, jc: (0, 0)),
            pl.BlockSpec((1, cout), lambda b, jc: (0, 0)),
            pl.BlockSpec((1, cout), lambda b, jc: (0, 0)),
            pl.BlockSpec((1, _JB, N, 3, c8), lambda b, jc: (b, jc, 0, 0, 0)),
            pl.BlockSpec((1, N, 3, c8), lambda b, jc: (b, 0, 0, 0)),
            pl.BlockSpec((2 * c8, cout), lambda b, jc: (0, 0)),
            pl.BlockSpec((2 * c8, cout), lambda b, jc: (0, 0)),
        ],
        out_specs=pl.BlockSpec((1, N, 3, cpad), lambda b, jc: (b, 0, 0, 0)),
        out_shape=jax.ShapeDtypeStruct((B, N, 3, cpad), jnp.float32),
    )(stats, gam, bet, g, xr4, wf, wd)


def _mm5_body(x_ref, w_ref, p_ref):
    xb = x_ref[0].astype(jnp.bfloat16)
    p_ref[0] = lax.dot_general(xb, w_ref[...], (((1,), (0,)), ((), ())),
                               preferred_element_type=jnp.float32)


def _mm5(xr, w5, crow, segw):
    return pl.pallas_call(
        _mm5_body,
        grid=(B, 6),
        in_specs=[
            pl.BlockSpec((1, N, crow), lambda b, s: (b, 0, 0)),
            pl.BlockSpec((crow, segw), lambda b, s: (0, s)),
        ],
        out_specs=pl.BlockSpec((1, N, segw), lambda b, s: (b, 0, s)),
        out_shape=jax.ShapeDtypeStruct((B, N, 6 * segw), jnp.float32),
    )(xr, w5)


def _stats5_body(g_ref, s_ref, *, segw, cnt):
    ph = pl.program_id(0)
    first = (pl.program_id(1) == 0) & (pl.program_id(2) == 0)

    @pl.when((ph == 0) & first)
    def _():
        s_ref[...] = jnp.zeros_like(s_ref)

    g = g_ref[0]
    nsq = None
    for d in range(3):
        p = g[:, d * segw:(d + 1) * segw]
        nsq = p * p if nsq is None else nsq + p * p
    norm = jnp.sqrt(nsq) + EPS

    @pl.when(ph == 0)
    def _():
        s_ref[0, :] += jnp.sum(norm, axis=0)

    @pl.when(ph == 1)
    def _():
        mean = s_ref[0, :] / cnt
        dev = norm - mean[None, :]
        s_ref[1, :] += jnp.sum(dev * dev, axis=0)


def _stats5(p5, segw):
    return pl.pallas_call(
        functools.partial(_stats5_body, segw=segw, cnt=B * N),
        grid=(2, B, N // 512),
        in_specs=[pl.BlockSpec((1, 512, 6 * segw),
                               lambda ph, b, nb: (b, nb, 0))],
        out_specs=pl.BlockSpec((2, segw), lambda ph, b, nb: (0, 0)),
        out_shape=jax.ShapeDtypeStruct((2, segw), jnp.float32),
    )(p5)


def _apply5_body(s_ref, gam_ref, bet_ref, g_ref, o_ref, *, segw, cnt):
    nb = pl.program_id(1)
    mean = s_ref[0, :] / cnt
    var = s_ref[1, :] / cnt
    istd = jnp.sqrt(var + 1e-5)
    gam = gam_ref[0]
    bet = bet_ref[0]

    g = g_ref[0]
    p = [g[:, d * segw:(d + 1) * segw] for d in range(3)]
    dd = [g[:, (3 + d) * segw:(4 + d) * segw] for d in range(3)]
    nsq = p[0] * p[0] + p[1] * p[1] + p[2] * p[2]
    norm = jnp.sqrt(nsq) + EPS
    nbn = (norm - mean[None, :]) / istd[None, :] * gam[None, :] + bet[None, :]
    pr = [pi / norm * nbn for pi in p]
    dot = pr[0] * dd[0] + pr[1] * dd[1] + pr[2] * dd[2]
    dnsq = dd[0] * dd[0] + dd[1] * dd[1] + dd[2] * dd[2]
    coef = jnp.where(dot >= 0.0, 0.0, dot / (dnsq + EPS))

    @pl.when(nb == 0)
    def _():
        o_ref[...] = jnp.zeros_like(o_ref)

    for d in range(3):
        od = pr[d] - coef * dd[d]
        o_ref[0, d, :] += jnp.sum(od, axis=0)

    @pl.when(nb == N // 512 - 1)
    def _():
        o_ref[...] *= (1.0 / N)


def _apply5(stats, gam, bet, p5, segw):
    return pl.pallas_call(
        functools.partial(_apply5_body, segw=segw, cnt=B * N),
        grid=(B, N // 512),
        in_specs=[
            pl.BlockSpec((2, segw), lambda b, nb: (0, 0)),
            pl.BlockSpec((1, segw), lambda b, nb: (0, 0)),
            pl.BlockSpec((1, segw), lambda b, nb: (0, 0)),
            pl.BlockSpec((1, 512, 6 * segw), lambda b, nb: (b, nb, 0)),
        ],
        out_specs=pl.BlockSpec((1, 3, segw), lambda b, nb: (b, 0, 0)),
        out_shape=jax.ShapeDtypeStruct((B, 3, segw), jnp.float32),
    )(stats, gam, bet, p5)


def _blockdiag(wt, crow, c, segw):
    cout = wt.shape[1]
    m = jnp.zeros((crow, 3 * segw), dtype=wt.dtype)
    for d in range(3):
        m = m.at[d * c:(d + 1) * c, d * segw:d * segw + cout].set(wt)
    return m


def _prep_w2(w, c8):
    c = w.shape[1] // 2
    wa = jnp.pad(w[:, :c].T, ((0, c8 - c), (0, 0)))
    wb = jnp.pad(w[:, c:].T, ((0, c8 - c), (0, 0)))
    return jnp.concatenate([wa, wb], axis=0).astype(jnp.bfloat16)


def _conv_layer(xr4, xt, wf, wd, gam, bet, cout, cpad):
    c8 = xr4.shape[-1]
    crow = 3 * c8
    idx, idxg = _knn(xt, xt.shape[1])
    wfp = _prep_w2(wf, c8)
    wdp = _prep_w2(wd, c8)
    if crow % 128 == 0:
        g = _gather_sc(idxg.reshape(_SC_P), xr4.reshape(B * N, crow), crow)
        g = g.reshape(B, K, N, 3, c8)
    else:
        g = _gather(idx, xr4.reshape(B, N, crow), crow).reshape(B, K, N, 3, c8)
    stats = _stats(g, xr4, wfp, c8, cout)
    out = _apply(stats, gam.reshape(1, cout), bet.reshape(1, cout),
                 g, xr4, wfp, wdp, c8, cout, cpad)
    xt_next = jnp.transpose(out, (0, 3, 2, 1)).reshape(B, cpad * 3, N)[:, :cout * 3]
    return out, xt_next


def kernel(x, W1f, W1d, g1, b1, W2f, W2d, g2, b2, W3f, W3d, g3, b3,
           W4f, W4d, g4, b4, W5f, W5d, g5, b5):
    xr0 = jnp.pad(jnp.transpose(x, (0, 2, 1))[..., None],
                  ((0, 0), (0, 0), (0, 0), (0, 7)))
    x1, t1 = _conv_layer(xr0, x, W1f, W1d, g1, b1, 64, 128)
    x2, t2 = _conv_layer(x1, t1, W2f, W2d, g2, b2, 64, 128)
    x3, t3 = _conv_layer(x2, t2, W3f, W3d, g3, b3, 128, 128)
    x4, _ = _conv_layer(x3, t3, W4f, W4d, g4, b4, 256, 256)
    xc = jnp.concatenate([x1[..., :64], x2[..., :64], x3, x4],
                         axis=-1).reshape(B, N, 3 * 512)
    w5 = jnp.concatenate(
        [_blockdiag(W5f.T, 3 * 512, 512, 1024),
         _blockdiag(W5d.T, 3 * 512, 512, 1024)], axis=1).astype(jnp.bfloat16)
    p5 = _mm5(xc, w5, 3 * 512, 1024)
    st5 = _stats5(p5, 1024)
    o5 = _apply5(st5, g5.reshape(1, 1024), b5.reshape(1, 1024), p5, 1024)
    return jnp.transpose(o5, (0, 2, 1))
